# Initial kernel scaffold; baseline (speedup 1.0000x reference)
import jax
import jax.numpy as jnp
from jax.experimental import pallas as pl


def kernel(hidden_states, W, b):
    raise NotImplementedError("write your pallas kernel here")


# speedup vs baseline: 1.0014x; 1.0014x over previous
import jax
import jax.numpy as jnp
from jax import lax
from jax.experimental import pallas as pl

HIDDEN = 1024
NUM_EXPERTS = 8
CAP = 5120
SCHUNK = 2048


def _softmax_body(x_ref, w_ref, b_ref, out_ref):
    x = x_ref[0]
    w = w_ref[...]
    z = lax.dot_general(w, x, (((1,), (1,)), ((), ())),
                        preferred_element_type=jnp.float32)
    z = z + b_ref[...]
    m = jnp.max(z, axis=0, keepdims=True)
    e = jnp.exp(z - m)
    s = jnp.sum(e, axis=0, keepdims=True)
    out_ref[0] = e / s


def _router_weights(hidden_states, W, b):
    B, S, H = hidden_states.shape
    b2 = b.reshape(NUM_EXPERTS, 1)
    return pl.pallas_call(
        _softmax_body,
        grid=(B, S // SCHUNK),
        in_specs=[
            pl.BlockSpec((1, SCHUNK, H), lambda i, j: (i, j, 0)),
            pl.BlockSpec((NUM_EXPERTS, H), lambda i, j: (0, 0)),
            pl.BlockSpec((NUM_EXPERTS, 1), lambda i, j: (0, 0)),
        ],
        out_specs=pl.BlockSpec((1, NUM_EXPERTS, SCHUNK), lambda i, j: (i, 0, j)),
        out_shape=jax.ShapeDtypeStruct((B, NUM_EXPERTS, S), jnp.float32),
    )(hidden_states, W, b2)


def kernel(hidden_states, W, b):
    weights = _router_weights(hidden_states, W, b)
    vals, idx = lax.top_k(weights, CAP)
    top_weights = jnp.transpose(vals, (0, 2, 1))
    indices = jnp.transpose(idx, (0, 2, 1)).astype(jnp.int64)
    return top_weights, indices


# speedup vs baseline: 1.0270x; 1.0256x over previous
import functools

import jax
import jax.numpy as jnp
from jax import lax
from jax.experimental import pallas as pl
from jax.experimental.pallas import tpu as pltpu
from jax.experimental.pallas import tpu_sc as plsc

HIDDEN = 1024
NUM_EXPERTS = 8
CAP = 5120
SCHUNK = 2048
S = 8192
ROWS = 32
NVREG = S // 16


def _softmax_body(x_ref, w_ref, b_ref, out_ref):
    x = x_ref[0]
    w = w_ref[...]
    z = lax.dot_general(w, x, (((1,), (1,)), ((), ())),
                        preferred_element_type=jnp.float32)
    z = z + b_ref[...]
    m = jnp.max(z, axis=0, keepdims=True)
    e = jnp.exp(z - m)
    s = jnp.sum(e, axis=0, keepdims=True)
    out_ref[0] = e / s


def _router_weights(hidden_states, W, b):
    B, Sv, H = hidden_states.shape
    b2 = b.reshape(NUM_EXPERTS, 1)
    return pl.pallas_call(
        _softmax_body,
        grid=(B, Sv // SCHUNK),
        in_specs=[
            pl.BlockSpec((1, SCHUNK, H), lambda i, j: (i, j, 0)),
            pl.BlockSpec((NUM_EXPERTS, H), lambda i, j: (0, 0)),
            pl.BlockSpec((NUM_EXPERTS, 1), lambda i, j: (0, 0)),
        ],
        out_specs=pl.BlockSpec((1, NUM_EXPERTS, SCHUNK), lambda i, j: (i, 0, j)),
        out_shape=jax.ShapeDtypeStruct((B, NUM_EXPERTS, Sv), jnp.float32),
    )(hidden_states, W, b2)


def _make_sort_kernel():
    mesh = plsc.VectorSubcoreMesh(core_axis_name="c", subcore_axis_name="s")
    num_cores = 2

    @functools.partial(
        pl.kernel,
        mesh=mesh,
        compiler_params=pltpu.CompilerParams(needs_layout_passes=False),
        out_type=(
            jax.ShapeDtypeStruct((ROWS, CAP), jnp.float32),
            jax.ShapeDtypeStruct((ROWS, CAP), jnp.int32),
        ),
        scratch_types=[
            pltpu.VMEM((S,), jnp.float32),
            pltpu.VMEM((S,), jnp.float32),
            pltpu.VMEM((S,), jnp.int32),
            pltpu.VMEM((S,), jnp.int32),
            pltpu.VMEM((32,), jnp.int32),
        ],
    )
    def sort_rows(w_hbm, outv_hbm, outi_hbm, ka, kb, ia, ib, rn):
        wid = lax.axis_index("s") * num_cores + lax.axis_index("c")
        lane = lax.iota(jnp.int32, 16)
        zeros16 = jnp.zeros((16,), jnp.int32)

        probe, _ = plsc.scan_count(zeros16)
        occ_base = jnp.min(probe)

        pltpu.sync_copy(w_hbm.at[wid], ka)

        bufs = [(ka, ia), (kb, ib)]
        for p in range(6):
            sh = 5 * p
            src_k, src_i = bufs[p % 2]
            dst_k, dst_i = bufs[(p + 1) % 2]

            rn[pl.ds(0, 16)] = zeros16
            rn[pl.ds(16, 16)] = zeros16

            def hist_body(j, c, src_k=src_k, sh=sh):
                v = src_k[pl.ds(j * 16, 16)]
                d = lax.shift_right_logical(plsc.bitcast(v, jnp.int32), sh) & 31
                occ, last = plsc.scan_count(d)
                plsc.addupdate_scatter(rn, [d], occ - occ_base + 1, mask=last)
                return c

            lax.fori_loop(0, NVREG, hist_body, 0, unroll=4)

            h0 = rn[pl.ds(0, 16)]
            h1 = rn[pl.ds(16, 16)]
            r1 = lax.rev(h1, (0,))
            r0 = lax.rev(h0, (0,))
            c1 = plsc.cumsum(r1)
            c0 = plsc.cumsum(r0)
            tot_hi = jnp.max(c1)
            ex1 = c1 - r1
            ex0 = (c0 - r0) + tot_hi
            rn[pl.ds(0, 16)] = lax.rev(ex0, (0,))
            rn[pl.ds(16, 16)] = lax.rev(ex1, (0,))

            def perm_body(j, c, src_k=src_k, src_i=src_i,
                          dst_k=dst_k, dst_i=dst_i, sh=sh, first=(p == 0)):
                v = src_k[pl.ds(j * 16, 16)]
                iv = (j * 16 + lane) if first else src_i[pl.ds(j * 16, 16)]
                d = lax.shift_right_logical(plsc.bitcast(v, jnp.int32), sh) & 31
                occ, last = plsc.scan_count(d)
                occ0 = occ - occ_base
                base = plsc.load_gather(rn, [d])
                pos = base + occ0
                plsc.addupdate_scatter(rn, [d], occ0 + 1, mask=last)
                plsc.store_scatter(dst_k, [pos], v)
                plsc.store_scatter(dst_i, [pos], iv)
                return c

            lax.fori_loop(0, NVREG, perm_body, 0, unroll=4)

        pltpu.sync_copy(ka.at[pl.ds(0, CAP)], outv_hbm.at[wid])
        pltpu.sync_copy(ia.at[pl.ds(0, CAP)], outi_hbm.at[wid])

    return sort_rows


_sort_rows = _make_sort_kernel()


def kernel(hidden_states, W, b):
    B = hidden_states.shape[0]
    weights = _router_weights(hidden_states, W, b)
    vals, idx = _sort_rows(weights.reshape(ROWS, S))
    top_weights = jnp.transpose(vals.reshape(B, NUM_EXPERTS, CAP), (0, 2, 1))
    indices = jnp.transpose(idx.reshape(B, NUM_EXPERTS, CAP), (0, 2, 1))
    return top_weights, indices.astype(jnp.int64)
